# baseline (speedup 1.0000x reference)
import jax
import jax.numpy as jnp
from jax.experimental import pallas as pl
from jax.experimental.pallas import tpu as pltpu

N = 50000
D = 512
G = 256
K = 5
SB = 1000
BLK = K * SB
NBLK = N // BLK

_NEG = -1e30


def _pool_body(*refs):
    feat_refs = refs[:K]
    seg_refs = refs[K:2 * K]
    gw_ref, fw_ref, fb_ref, out_ref, m_ref, s_ref, u_ref = refs[2 * K:]
    i = pl.program_id(0)

    @pl.when(i == 0)
    def _init():
        m_ref[0, 0] = jnp.float32(_NEG)
        s_ref[...] = jnp.zeros((1, G), jnp.float32)
        u_ref[...] = jnp.zeros((D, G), jnp.float32)

    gw_hi = gw_ref[...].astype(jnp.bfloat16)
    gw_lo = (gw_ref[...] - gw_hi.astype(jnp.float32)).astype(jnp.bfloat16)

    gates = []
    bm = jnp.float32(_NEG)
    for k in range(K):
        x = feat_refs[k][...]
        xb = x.astype(jnp.bfloat16)
        gate = (jnp.dot(xb, gw_hi, preferred_element_type=jnp.float32)
                + jnp.dot(xb, gw_lo, preferred_element_type=jnp.float32)
                + jnp.dot((x - xb.astype(jnp.float32)).astype(jnp.bfloat16),
                          gw_hi, preferred_element_type=jnp.float32))
        gates.append(gate)
        bm = jnp.maximum(bm, jnp.max(gate))

    m_old = m_ref[0, 0]
    m_new = jnp.maximum(m_old, bm)
    m_ref[0, 0] = m_new
    scale = jnp.exp(m_old - m_new)

    gids = jax.lax.broadcasted_iota(jnp.int32, (1, G), 1)
    ones_row = jnp.ones((1, SB), jnp.bfloat16)
    bs = jnp.zeros((1, G), jnp.float32)
    contrib = jnp.zeros((D, G), jnp.float32)
    for k in range(K):
        e = jnp.exp(jnp.maximum(gates[k] - m_new, -80.0))
        mask_bf = (seg_refs[k][...] == gids).astype(jnp.bfloat16)
        we = mask_bf * e.astype(jnp.bfloat16)
        bs = bs + jax.lax.dot_general(ones_row, we, (((1,), (0,)), ((), ())),
                                      preferred_element_type=jnp.float32)
        contrib = contrib + jax.lax.dot_general(
            feat_refs[k][...].astype(jnp.bfloat16), we,
            (((0,), (0,)), ((), ())),
            preferred_element_type=jnp.float32)

    s_ref[...] = s_ref[...] * scale + bs
    u_ref[...] = u_ref[...] * scale + contrib

    @pl.when(i == NBLK - 1)
    def _finish():
        s = s_ref[...]
        nonempty = s > 0.0
        pooled_t = jnp.where(nonempty, u_ref[...] / s, 0.0)
        out = jax.lax.dot_general(pooled_t, fw_ref[...], (((0,), (0,)), ((), ())),
                                  preferred_element_type=jnp.float32)
        ind = jnp.transpose(nonempty.astype(jnp.float32))
        out_ref[...] = out + fb_ref[...] * ind


@jax.jit
def _pool(feat, seg, gate_w, feat_w, feat_b):
    feat_specs = [pl.BlockSpec((SB, D), lambda i, k=k: (K * i + k, 0))
                  for k in range(K)]
    seg_specs = [pl.BlockSpec((SB, 1), lambda i, k=k: (K * i + k, 0))
                 for k in range(K)]
    return pl.pallas_call(
        _pool_body,
        grid=(NBLK,),
        in_specs=feat_specs + seg_specs + [
            pl.BlockSpec((D, 1), lambda i: (0, 0)),
            pl.BlockSpec((D, D), lambda i: (0, 0)),
            pl.BlockSpec((1, D), lambda i: (0, 0)),
        ],
        out_specs=pl.BlockSpec((G, D), lambda i: (0, 0)),
        out_shape=jax.ShapeDtypeStruct((G, D), jnp.float32),
        scratch_shapes=[
            pltpu.SMEM((1, 1), jnp.float32),
            pltpu.VMEM((1, G), jnp.float32),
            pltpu.VMEM((D, G), jnp.float32),
        ],
    )(*([feat] * K), *([seg] * K), gate_w, feat_w, feat_b)


def kernel(feat, gate_w, gate_b, feat_w, feat_b, segment_ids):
    del gate_b
    seg = segment_ids.astype(jnp.int32).reshape(N, 1)
    return _pool(feat, seg, gate_w, feat_w, feat_b.reshape(1, D))


# speedup vs baseline: 1.4320x; 1.4320x over previous
import jax
import jax.numpy as jnp
from jax.experimental import pallas as pl
from jax.experimental.pallas import tpu as pltpu

N = 50000
D = 512
G = 256
K = 5
SB = 1000
BLK = K * SB
NBLK = N // BLK

_NEG = -(2.0 ** 100)


def _pool_body(*refs):
    feat_refs = refs[:K]
    seg_refs = refs[K:2 * K]
    gw_ref, gb_ref, fw_ref, fb_ref, out_ref, m_ref, s_ref, u_ref = refs[2 * K:]
    i = pl.program_id(0)

    @pl.when(i == 0)
    def _init():
        m_ref[...] = jnp.full((1, G), _NEG, jnp.float32)
        s_ref[...] = jnp.zeros((1, G), jnp.float32)
        u_ref[...] = jnp.zeros((D, G), jnp.float32)

    gids = jax.lax.broadcasted_iota(jnp.int32, (1, G), 1)

    gates = []
    bm = jnp.full((1, G), _NEG, jnp.float32)
    for k in range(K):
        x = feat_refs[k][...]
        seg = seg_refs[k][...]
        gate = jnp.dot(x, gw_ref[...],
                       preferred_element_type=jnp.float32) + gb_ref[0, 0]
        mask = seg == gids
        gates.append(gate)
        bm = jnp.maximum(bm, jnp.max(jnp.where(mask, gate, _NEG),
                                     axis=0, keepdims=True))

    m_old = m_ref[...]
    m_new = jnp.maximum(m_old, bm).astype(jnp.bfloat16).astype(jnp.float32)
    m_ref[...] = m_new
    m_col = jnp.transpose(m_new.astype(jnp.bfloat16))
    scale = jnp.exp(m_old - m_new)

    ones_row = jnp.ones((1, SB), jnp.bfloat16)
    bs = jnp.zeros((1, G), jnp.float32)
    contrib = jnp.zeros((D, G), jnp.float32)
    for k in range(K):
        mask_bf = (seg_refs[k][...] == gids).astype(jnp.bfloat16)
        mrow = jax.lax.dot_general(mask_bf, m_col, (((1,), (0,)), ((), ())),
                                   preferred_element_type=jnp.float32)
        e = jnp.exp(gates[k] - mrow)
        we = mask_bf * e.astype(jnp.bfloat16)
        bs = bs + jax.lax.dot_general(ones_row, we, (((1,), (0,)), ((), ())),
                                      preferred_element_type=jnp.float32)
        contrib = contrib + jax.lax.dot_general(
            feat_refs[k][...].astype(jnp.bfloat16), we,
            (((0,), (0,)), ((), ())),
            preferred_element_type=jnp.float32)

    s_ref[...] = s_ref[...] * scale + bs
    u_ref[...] = u_ref[...] * scale + contrib

    @pl.when(i == NBLK - 1)
    def _finish():
        s = s_ref[...]
        nonempty = s > 0.0
        pooled_t = jnp.where(nonempty, u_ref[...] / s, 0.0)
        out = jax.lax.dot_general(pooled_t, fw_ref[...], (((0,), (0,)), ((), ())),
                                  preferred_element_type=jnp.float32)
        ind = jnp.transpose(nonempty.astype(jnp.float32))
        out_ref[...] = out + fb_ref[...] * ind


@jax.jit
def _pool(feat, seg, gate_w, gate_b, feat_w, feat_b):
    feat_specs = [pl.BlockSpec((SB, D), lambda i, k=k: (K * i + k, 0))
                  for k in range(K)]
    seg_specs = [pl.BlockSpec((SB, 1), lambda i, k=k: (K * i + k, 0))
                 for k in range(K)]
    return pl.pallas_call(
        _pool_body,
        grid=(NBLK,),
        in_specs=feat_specs + seg_specs + [
            pl.BlockSpec((D, 1), lambda i: (0, 0)),
            pl.BlockSpec((1, 1), lambda i: (0, 0)),
            pl.BlockSpec((D, D), lambda i: (0, 0)),
            pl.BlockSpec((1, D), lambda i: (0, 0)),
        ],
        out_specs=pl.BlockSpec((G, D), lambda i: (0, 0)),
        out_shape=jax.ShapeDtypeStruct((G, D), jnp.float32),
        scratch_shapes=[
            pltpu.VMEM((1, G), jnp.float32),
            pltpu.VMEM((1, G), jnp.float32),
            pltpu.VMEM((D, G), jnp.float32),
        ],
    )(*([feat] * K), *([seg] * K), gate_w, gate_b, feat_w, feat_b)


def kernel(feat, gate_w, gate_b, feat_w, feat_b, segment_ids):
    seg = segment_ids.astype(jnp.int32).reshape(N, 1)
    return _pool(feat, seg, gate_w,
                 gate_b.reshape(1, 1).astype(jnp.float32),
                 feat_w, feat_b.reshape(1, D))


# speedup vs baseline: 1.4965x; 1.0451x over previous
import jax
import jax.numpy as jnp
from jax.experimental import pallas as pl
from jax.experimental.pallas import tpu as pltpu

N = 50000
D = 512
G = 256
K = 5
SB = 1000
BLK = K * SB
NBLK = N // BLK

_NEG = -(2.0 ** 100)


def _pool_body(*refs):
    feat_refs = refs[:K]
    seg_refs = refs[K:2 * K]
    gw_ref, gb_ref, fw_ref, fb_ref, out_ref, m_ref, s_ref, u_ref = refs[2 * K:]
    i = pl.program_id(0)

    @pl.when(i == 0)
    def _init():
        m_ref[...] = jnp.full((1, G), _NEG, jnp.float32)
        s_ref[...] = jnp.zeros((1, G), jnp.float32)
        u_ref[...] = jnp.zeros((D, G), jnp.float32)

    gids = jax.lax.broadcasted_iota(jnp.int32, (1, G), 1)

    gates, masks = [], []
    bm = jnp.full((1, G), _NEG, jnp.float32)
    for k in range(K):
        x = feat_refs[k][...]
        seg = seg_refs[k][...]
        gate = jnp.dot(x, gw_ref[...],
                       preferred_element_type=jnp.float32) + gb_ref[0, 0]
        mask = seg == gids
        gates.append(gate)
        masks.append(mask)
        bm = jnp.maximum(bm, jnp.max(jnp.where(mask, gate, _NEG),
                                     axis=0, keepdims=True))

    m_old = m_ref[...]
    m_new = jnp.maximum(m_old, bm).astype(jnp.bfloat16).astype(jnp.float32)
    m_ref[...] = m_new
    m_col = jnp.transpose(m_new.astype(jnp.bfloat16))
    scale = jnp.exp(m_old - m_new)

    ones_row = jnp.ones((1, SB), jnp.bfloat16)
    bs = jnp.zeros((1, G), jnp.float32)
    contrib = jnp.zeros((D, G), jnp.float32)
    for k in range(K):
        mask_bf = masks[k].astype(jnp.bfloat16)
        mrow = jax.lax.dot_general(mask_bf, m_col, (((1,), (0,)), ((), ())),
                                   preferred_element_type=jnp.float32)
        e = jnp.exp(gates[k] - mrow)
        we = mask_bf * e.astype(jnp.bfloat16)
        bs = bs + jax.lax.dot_general(ones_row, we, (((1,), (0,)), ((), ())),
                                      preferred_element_type=jnp.float32)
        contrib = contrib + jax.lax.dot_general(
            feat_refs[k][...].astype(jnp.bfloat16), we,
            (((0,), (0,)), ((), ())),
            preferred_element_type=jnp.float32)

    s_ref[...] = s_ref[...] * scale + bs
    u_ref[...] = u_ref[...] * scale + contrib

    @pl.when(i == NBLK - 1)
    def _finish():
        s = s_ref[...]
        nonempty = s > 0.0
        pooled_t = jnp.where(nonempty, u_ref[...] / s, 0.0)
        out = jax.lax.dot_general(pooled_t, fw_ref[...], (((0,), (0,)), ((), ())),
                                  preferred_element_type=jnp.float32)
        ind = jnp.transpose(nonempty.astype(jnp.float32))
        out_ref[...] = out + fb_ref[...] * ind


@jax.jit
def _pool(feat, seg, gate_w, gate_b, feat_w, feat_b):
    feat_specs = [pl.BlockSpec((SB, D), lambda i, k=k: (K * i + k, 0))
                  for k in range(K)]
    seg_specs = [pl.BlockSpec((SB, 1), lambda i, k=k: (K * i + k, 0))
                 for k in range(K)]
    return pl.pallas_call(
        _pool_body,
        grid=(NBLK,),
        in_specs=feat_specs + seg_specs + [
            pl.BlockSpec((D, 1), lambda i: (0, 0)),
            pl.BlockSpec((1, 1), lambda i: (0, 0)),
            pl.BlockSpec((D, D), lambda i: (0, 0)),
            pl.BlockSpec((1, D), lambda i: (0, 0)),
        ],
        out_specs=pl.BlockSpec((G, D), lambda i: (0, 0)),
        out_shape=jax.ShapeDtypeStruct((G, D), jnp.float32),
        scratch_shapes=[
            pltpu.VMEM((1, G), jnp.float32),
            pltpu.VMEM((1, G), jnp.float32),
            pltpu.VMEM((D, G), jnp.float32),
        ],
    )(*([feat] * K), *([seg] * K), gate_w, gate_b, feat_w, feat_b)


def kernel(feat, gate_w, gate_b, feat_w, feat_b, segment_ids):
    seg = segment_ids.astype(jnp.int32).reshape(N, 1)
    return _pool(feat, seg, gate_w,
                 gate_b.reshape(1, 1).astype(jnp.float32),
                 feat_w, feat_b.reshape(1, D))
